# Initial kernel scaffold; baseline (speedup 1.0000x reference)
import functools

import jax
import jax.numpy as jnp
from jax import lax
from jax.experimental import pallas as pl
from jax.experimental.pallas import tpu as pltpu
from jax.experimental.pallas import tpu_sc as plsc

N, E, D, H = 10000, 320000, 128, 128
NC, NS = 2, 16
NW = NC * NS
ROWS = E // D
BASE_ROWS = ROWS // NW
EXTRA = ROWS - BASE_ROWS * NW
ROWS_PAD = ((ROWS + 1) + 7) // 8 * 8
NPT = N // NS

_mesh = functools.partial(
    plsc.VectorSubcoreMesh,
    core_axis_name="c", subcore_axis_name="s", num_cores=NC, num_subcores=NS,
)


def _worker_range(wid):
    nrows = BASE_ROWS + jnp.where(wid < EXTRA, 1, 0).astype(jnp.int32)
    base_row = BASE_ROWS * wid + jnp.minimum(wid, EXTRA)
    return base_row, nrows


def _gather_body(ps_hbm, pr_hbm, sidx_hbm, ridx_hbm, gs_hbm, gr_hbm,
                 idx_v, rows_v, sem):
    wid = (lax.axis_index("s") * NC + lax.axis_index("c")).astype(jnp.int32)
    base_row, nrows = _worker_range(wid)
    for tab, idx_hbm, out in ((ps_hbm, sidx_hbm, gs_hbm),
                              (pr_hbm, ridx_hbm, gr_hbm)):
        pltpu.sync_copy(idx_hbm.at[pl.ds(base_row, BASE_ROWS + 1)], idx_v)

        def body(j, carry, tab=tab, out=out):
            pltpu.async_copy(tab.at[idx_v.at[j]], rows_v, sem).wait()
            pltpu.sync_copy(rows_v, out.at[pl.ds((base_row + j) * D, D)])
            return carry

        lax.fori_loop(0, nrows, body, 0)


def _scatter_body(u_hbm, ridx_hbm, out_hbm, idx_v, ubuf, zbuf, obuf, agg_sh):
    c = lax.axis_index("c").astype(jnp.int32)
    s = lax.axis_index("s").astype(jnp.int32)
    wid = s * NC + c
    zero = jnp.zeros((16,), jnp.float32)
    for i in range(zbuf.shape[0]):
        for l in range(D // 16):
            zbuf[i, pl.ds(l * 16, 16)] = zero
    for k in range(NPT // zbuf.shape[0]):
        pltpu.sync_copy(zbuf, agg_sh.at[pl.ds(s * NPT + k * zbuf.shape[0],
                                              zbuf.shape[0])])
    plsc.subcore_barrier()
    base_row, nrows = _worker_range(wid)
    pltpu.sync_copy(ridx_hbm.at[pl.ds(base_row, BASE_ROWS + 1)], idx_v)

    def body(j, carry):
        pltpu.sync_copy(u_hbm.at[pl.ds((base_row + j) * D, D)], ubuf)
        pltpu.sync_copy(ubuf, agg_sh.at[idx_v.at[j]], add=True)
        return carry

    lax.fori_loop(0, nrows, body, 0)
    plsc.subcore_barrier()
    ob = obuf.shape[0]
    for k in range(NPT // ob):
        off = s * NPT + k * ob
        pltpu.sync_copy(agg_sh.at[pl.ds(off, ob)], obuf)
        pltpu.sync_copy(obuf, out_hbm.at[c].at[pl.ds(off, ob)])


def _proj_body(node_ref, ws_ref, wr_ref, ps_ref, pr_ref):
    x = node_ref[...]
    ps_ref[...] = jnp.dot(x, ws_ref[...], preferred_element_type=jnp.float32)
    pr_ref[...] = jnp.dot(x, wr_ref[...], preferred_element_type=jnp.float32)


def _edge_body(gs_ref, gr_ref, ef_ref, we_ref, we2_ref, b1_ref, b2_ref,
               u_ref, ne_ref):
    ef = ef_ref[...]
    x = gs_ref[...] + gr_ref[...] + b1_ref[...]
    x = x + jnp.dot(ef, we_ref[...], preferred_element_type=jnp.float32)
    h = jnp.maximum(x, 0.0)
    u = jnp.dot(h, we2_ref[...], preferred_element_type=jnp.float32) + b2_ref[...]
    u_ref[...] = u
    ne_ref[...] = u + ef


def _node_body(nf_ref, p0_ref, p1_ref, w1a_ref, w1b_ref, w2_ref, b1_ref,
               b2_ref, out_ref):
    nf = nf_ref[...]
    agg = p0_ref[...] + p1_ref[...]
    x = (jnp.dot(nf, w1a_ref[...], preferred_element_type=jnp.float32)
         + jnp.dot(agg, w1b_ref[...], preferred_element_type=jnp.float32)
         + b1_ref[...])
    h = jnp.maximum(x, 0.0)
    out_ref[...] = (jnp.dot(h, w2_ref[...], preferred_element_type=jnp.float32)
                    + b2_ref[...] + nf)


_BN = 1000
_BE = 2000


def _full(i):
    return (0, 0)


def _rowblk(i):
    return (i, 0)


def kernel(node_features, edge_features, senders, receivers,
           We1, be1, We2, be2, Wn1, bn1, Wn2, bn2):
    f32 = jnp.float32
    pad = ROWS_PAD * D - E
    sidx = jnp.pad(senders.astype(jnp.int32), (0, pad)).reshape(ROWS_PAD, D)
    ridx = jnp.pad(receivers.astype(jnp.int32), (0, pad)).reshape(ROWS_PAD, D)
    Ws, Wr, We = We1[:D], We1[D:2 * D], We1[2 * D:]
    Wn1a, Wn1b = Wn1[:D], Wn1[D:]
    b_e1, b_e2 = be1.reshape(1, H), be2.reshape(1, D)
    b_n1, b_n2 = bn1.reshape(1, H), bn2.reshape(1, D)

    ps, pr = pl.pallas_call(
        _proj_body,
        grid=(N // _BN,),
        in_specs=[pl.BlockSpec((_BN, D), _rowblk),
                  pl.BlockSpec((D, D), _full),
                  pl.BlockSpec((D, D), _full)],
        out_specs=[pl.BlockSpec((_BN, D), _rowblk)] * 2,
        out_shape=[jax.ShapeDtypeStruct((N, D), f32)] * 2,
    )(node_features, Ws, Wr)

    gather = pl.kernel(
        _gather_body,
        out_type=[jax.ShapeDtypeStruct((E, D), f32)] * 2,
        mesh=_mesh(),
        scratch_types=[pltpu.VMEM((BASE_ROWS + 1, D), jnp.int32),
                       pltpu.VMEM((D, D), f32),
                       pltpu.SemaphoreType.DMA],
    )
    gs, gr = gather(ps, pr, sidx, ridx)

    u, new_edge = pl.pallas_call(
        _edge_body,
        grid=(E // _BE,),
        in_specs=[pl.BlockSpec((_BE, D), _rowblk),
                  pl.BlockSpec((_BE, D), _rowblk),
                  pl.BlockSpec((_BE, D), _rowblk),
                  pl.BlockSpec((D, H), _full),
                  pl.BlockSpec((H, D), _full),
                  pl.BlockSpec((1, H), _full),
                  pl.BlockSpec((1, D), _full)],
        out_specs=[pl.BlockSpec((_BE, D), _rowblk)] * 2,
        out_shape=[jax.ShapeDtypeStruct((E, D), f32)] * 2,
    )(gs, gr, edge_features, We, We2, b_e1, b_e2)

    scatter = pl.kernel(
        _scatter_body,
        out_type=jax.ShapeDtypeStruct((NC, N, D), f32),
        mesh=_mesh(),
        scratch_types=[pltpu.VMEM((BASE_ROWS + 1, D), jnp.int32),
                       pltpu.VMEM((D, D), f32),
                       pltpu.VMEM((25, D), f32),
                       pltpu.VMEM((125, D), f32),
                       pltpu.VMEM_SHARED((N, D), f32)],
    )
    parts = scatter(u, ridx)

    new_node = pl.pallas_call(
        _node_body,
        grid=(N // _BN,),
        in_specs=[pl.BlockSpec((_BN, D), _rowblk),
                  pl.BlockSpec((_BN, D), _rowblk),
                  pl.BlockSpec((_BN, D), _rowblk),
                  pl.BlockSpec((D, H), _full),
                  pl.BlockSpec((D, H), _full),
                  pl.BlockSpec((H, D), _full),
                  pl.BlockSpec((1, H), _full),
                  pl.BlockSpec((1, D), _full)],
        out_specs=pl.BlockSpec((_BN, D), _rowblk),
        out_shape=jax.ShapeDtypeStruct((N, D), f32),
    )(node_features, parts[0], parts[1], Wn1a, Wn1b, Wn2, b_n1, b_n2)

    return new_node, new_edge


# speedup vs baseline: 3.0956x; 3.0956x over previous
import functools

import jax
import jax.numpy as jnp
from jax import lax
from jax.experimental import pallas as pl
from jax.experimental.pallas import tpu as pltpu
from jax.experimental.pallas import tpu_sc as plsc

N, E, D, H = 10000, 320000, 128, 128
NC, NS = 2, 16
NW = NC * NS
ROWS = E // D
BASE_ROWS = ROWS // NW
EXTRA = ROWS - BASE_ROWS * NW
IDX_STAGE = 88
ROWS_PAD = 2416 + IDX_STAGE
N_PAD = 10240
NPT = N_PAD // NS

_mesh = functools.partial(
    plsc.VectorSubcoreMesh,
    core_axis_name="c", subcore_axis_name="s", num_cores=NC, num_subcores=NS,
)


def _worker_range(wid):
    nrows = BASE_ROWS + jnp.where(wid < EXTRA, 1, 0).astype(jnp.int32)
    base_row = BASE_ROWS * wid + jnp.minimum(wid, EXTRA)
    aligned = (base_row // 8) * 8
    return base_row, nrows, aligned, base_row - aligned


def _gather_body(ps_hbm, pr_hbm, sidx_hbm, ridx_hbm, gs_hbm, gr_hbm,
                 idx_v, rows_v, sem):
    wid = (lax.axis_index("s") * NC + lax.axis_index("c")).astype(jnp.int32)
    base_row, nrows, aligned, skew = _worker_range(wid)
    for tab, idx_hbm, out in ((ps_hbm, sidx_hbm, gs_hbm),
                              (pr_hbm, ridx_hbm, gr_hbm)):
        pltpu.sync_copy(idx_hbm.at[pl.ds(aligned, IDX_STAGE)], idx_v)

        def body(j, carry, tab=tab, out=out):
            pltpu.async_copy(tab.at[idx_v.at[skew + j]], rows_v, sem).wait()
            pltpu.sync_copy(rows_v, out.at[pl.ds((base_row + j) * D, D)])
            return carry

        lax.fori_loop(0, nrows, body, 0)


def _scatter_body(u_hbm, ridx_hbm, out_hbm, idx_v, ubuf, zbuf, obuf, agg_sh):
    c = lax.axis_index("c").astype(jnp.int32)
    s = lax.axis_index("s").astype(jnp.int32)
    wid = s * NC + c
    zero = jnp.zeros((16,), jnp.float32)
    zb = zbuf.shape[0]
    for i in range(zb):
        for l in range(D // 16):
            zbuf[i, pl.ds(l * 16, 16)] = zero
    for k in range(NPT // zb):
        pltpu.sync_copy(zbuf, agg_sh.at[pl.ds(s * NPT + k * zb, zb)])
    plsc.subcore_barrier()
    base_row, nrows, aligned, skew = _worker_range(wid)
    pltpu.sync_copy(ridx_hbm.at[pl.ds(aligned, IDX_STAGE)], idx_v)

    def body(j, carry):
        pltpu.sync_copy(u_hbm.at[pl.ds((base_row + j) * D, D)], ubuf)
        pltpu.sync_copy(ubuf, agg_sh.at[idx_v.at[skew + j]], add=True)
        return carry

    lax.fori_loop(0, nrows, body, 0)
    plsc.subcore_barrier()
    ob = obuf.shape[0]
    for k in range(NPT // ob):
        off = s * NPT + k * ob
        pltpu.sync_copy(agg_sh.at[pl.ds(off, ob)], obuf)
        pltpu.sync_copy(obuf, out_hbm.at[c].at[pl.ds(off, ob)])


def _proj_body(node_ref, ws_ref, wr_ref, ps_ref, pr_ref):
    x = node_ref[...]
    ps_ref[...] = jnp.dot(x, ws_ref[...], preferred_element_type=jnp.float32)
    pr_ref[...] = jnp.dot(x, wr_ref[...], preferred_element_type=jnp.float32)


def _edge_body(gs_ref, gr_ref, ef_ref, we_ref, we2_ref, b1_ref, b2_ref,
               u_ref, ne_ref):
    ef = ef_ref[...]
    x = gs_ref[...] + gr_ref[...] + b1_ref[...]
    x = x + jnp.dot(ef, we_ref[...], preferred_element_type=jnp.float32)
    h = jnp.maximum(x, 0.0)
    u = jnp.dot(h, we2_ref[...], preferred_element_type=jnp.float32) + b2_ref[...]
    u_ref[...] = u
    ne_ref[...] = u + ef


def _node_body(nf_ref, p0_ref, p1_ref, w1a_ref, w1b_ref, w2_ref, b1_ref,
               b2_ref, out_ref):
    nf = nf_ref[...]
    agg = p0_ref[...] + p1_ref[...]
    x = (jnp.dot(nf, w1a_ref[...], preferred_element_type=jnp.float32)
         + jnp.dot(agg, w1b_ref[...], preferred_element_type=jnp.float32)
         + b1_ref[...])
    h = jnp.maximum(x, 0.0)
    out_ref[...] = (jnp.dot(h, w2_ref[...], preferred_element_type=jnp.float32)
                    + b2_ref[...] + nf)


_BN = 1000
_BE = 2000


def _full(i):
    return (0, 0)


def _rowblk(i):
    return (i, 0)


def kernel(node_features, edge_features, senders, receivers,
           We1, be1, We2, be2, Wn1, bn1, Wn2, bn2):
    f32 = jnp.float32
    pad = ROWS_PAD * D - E
    sidx = jnp.pad(senders.astype(jnp.int32), (0, pad)).reshape(ROWS_PAD, D)
    ridx = jnp.pad(receivers.astype(jnp.int32), (0, pad)).reshape(ROWS_PAD, D)
    Ws, Wr, We = We1[:D], We1[D:2 * D], We1[2 * D:]
    Wn1a, Wn1b = Wn1[:D], Wn1[D:]
    b_e1, b_e2 = be1.reshape(1, H), be2.reshape(1, D)
    b_n1, b_n2 = bn1.reshape(1, H), bn2.reshape(1, D)

    ps, pr = pl.pallas_call(
        _proj_body,
        grid=(N // _BN,),
        in_specs=[pl.BlockSpec((_BN, D), _rowblk),
                  pl.BlockSpec((D, D), _full),
                  pl.BlockSpec((D, D), _full)],
        out_specs=[pl.BlockSpec((_BN, D), _rowblk)] * 2,
        out_shape=[jax.ShapeDtypeStruct((N, D), f32)] * 2,
    )(node_features, Ws, Wr)

    gather = pl.kernel(
        _gather_body,
        out_type=[jax.ShapeDtypeStruct((E, D), f32)] * 2,
        mesh=_mesh(),
        scratch_types=[pltpu.VMEM((IDX_STAGE, D), jnp.int32),
                       pltpu.VMEM((D, D), f32),
                       pltpu.SemaphoreType.DMA],
    )
    gs, gr = gather(ps, pr, sidx, ridx)

    u, new_edge = pl.pallas_call(
        _edge_body,
        grid=(E // _BE,),
        in_specs=[pl.BlockSpec((_BE, D), _rowblk),
                  pl.BlockSpec((_BE, D), _rowblk),
                  pl.BlockSpec((_BE, D), _rowblk),
                  pl.BlockSpec((D, H), _full),
                  pl.BlockSpec((H, D), _full),
                  pl.BlockSpec((1, H), _full),
                  pl.BlockSpec((1, D), _full)],
        out_specs=[pl.BlockSpec((_BE, D), _rowblk)] * 2,
        out_shape=[jax.ShapeDtypeStruct((E, D), f32)] * 2,
    )(gs, gr, edge_features, We, We2, b_e1, b_e2)

    scatter = pl.kernel(
        _scatter_body,
        out_type=jax.ShapeDtypeStruct((NC, N_PAD, D), f32),
        mesh=_mesh(),
        scratch_types=[pltpu.VMEM((IDX_STAGE, D), jnp.int32),
                       pltpu.VMEM((D, D), f32),
                       pltpu.VMEM((32, D), f32),
                       pltpu.VMEM((D, D), f32),
                       pltpu.VMEM_SHARED((N_PAD, D), f32)],
    )
    parts = scatter(u, ridx)
    p0, p1 = parts[0, :N], parts[1, :N]

    new_node = pl.pallas_call(
        _node_body,
        grid=(N // _BN,),
        in_specs=[pl.BlockSpec((_BN, D), _rowblk),
                  pl.BlockSpec((_BN, D), _rowblk),
                  pl.BlockSpec((_BN, D), _rowblk),
                  pl.BlockSpec((D, H), _full),
                  pl.BlockSpec((D, H), _full),
                  pl.BlockSpec((H, D), _full),
                  pl.BlockSpec((1, H), _full),
                  pl.BlockSpec((1, D), _full)],
        out_specs=pl.BlockSpec((_BN, D), _rowblk),
        out_shape=jax.ShapeDtypeStruct((N, D), f32),
    )(node_features, p0, p1, Wn1a, Wn1b, Wn2, b_n1, b_n2)

    return new_node, new_edge


# speedup vs baseline: 3.8791x; 1.2531x over previous
import functools

import jax
import jax.numpy as jnp
from jax import lax
from jax.experimental import pallas as pl
from jax.experimental.pallas import tpu as pltpu
from jax.experimental.pallas import tpu_sc as plsc

N, E, D, H = 10000, 320000, 128, 128
NC, NS = 2, 16
NW = NC * NS
CHUNK = 128
ROWS2 = 2560
E_PAD = ROWS2 * CHUNK
CPW = ROWS2 // NW
PAIRS = CPW // 4
HPAIRS = CPW // 2
N_PAD = 10240
NPT = N_PAD // NS

_mesh = functools.partial(
    plsc.VectorSubcoreMesh,
    core_axis_name="c", subcore_axis_name="s", num_cores=NC, num_subcores=NS,
)


def _wid():
    return (lax.axis_index("s") * NC + lax.axis_index("c")).astype(jnp.int32)


def _gather_body(ps_hbm, pr_hbm, sidx_hbm, ridx_hbm, gs_hbm, gr_hbm,
                 idx_v, b0, b1, gsem, osem0, osem1):
    wid = _wid()
    base_c = wid * CPW
    for tab, idx_hbm, out in ((ps_hbm, sidx_hbm, gs_hbm),
                              (pr_hbm, ridx_hbm, gr_hbm)):
        pltpu.sync_copy(idx_hbm.at[pl.ds(base_c, CPW)], idx_v)

        def pair(p, carry, tab=tab, out=out):
            c0 = p * 4
            for buf, osem, cb in ((b0, osem0, c0), (b1, osem1, c0 + 2)):
                dst = out.at[pl.ds((base_c + cb) * CHUNK, 2 * CHUNK)]

                @pl.when(p > 0)
                def _(buf=buf, osem=osem, dst=dst):
                    pltpu.make_async_copy(dst, buf, osem).wait()

                d0 = pltpu.async_copy(tab.at[idx_v.at[cb]],
                                      buf.at[pl.ds(0, CHUNK)], gsem)
                d1 = pltpu.async_copy(tab.at[idx_v.at[cb + 1]],
                                      buf.at[pl.ds(CHUNK, CHUNK)], gsem)
                d0.wait()
                d1.wait()
                pltpu.async_copy(buf, dst, osem)
            return carry

        lax.fori_loop(0, PAIRS, pair, 0)
        pltpu.make_async_copy(out.at[pl.ds(0, 2 * CHUNK)], b0, osem0).wait()
        pltpu.make_async_copy(out.at[pl.ds(0, 2 * CHUNK)], b1, osem1).wait()


def _scatter_body(u_hbm, ridx_hbm, out_hbm, idx_v, ub0, ub1,
                  agg_sh, lsem0, lsem1):
    c = lax.axis_index("c").astype(jnp.int32)
    s = lax.axis_index("s").astype(jnp.int32)
    wid = s * NC + c
    zero = jnp.zeros((16,), jnp.float32)
    zb = 32
    for i in range(zb):
        for l in range(D // 16):
            ub0[i, pl.ds(l * 16, 16)] = zero
    zsrc = ub0.at[pl.ds(0, zb)]
    for k in range(NPT // zb):
        pltpu.sync_copy(zsrc, agg_sh.at[pl.ds(s * NPT + k * zb, zb)])
    plsc.subcore_barrier()
    base_c = wid * CPW
    pltpu.sync_copy(ridx_hbm.at[pl.ds(base_c, CPW)], idx_v)
    pltpu.async_copy(u_hbm.at[pl.ds(base_c * CHUNK, CHUNK)], ub0, lsem0)
    pltpu.async_copy(u_hbm.at[pl.ds((base_c + 1) * CHUNK, CHUNK)], ub1, lsem1)

    def pair(p, carry):
        for off, buf, lsem in ((0, ub0, lsem0), (1, ub1, lsem1)):
            cb = 2 * p + off
            src = u_hbm.at[pl.ds((base_c + cb) * CHUNK, CHUNK)]
            pltpu.make_async_copy(src, buf, lsem).wait()
            pltpu.sync_copy(buf, agg_sh.at[idx_v.at[cb]], add=True)

            @pl.when(p < HPAIRS - 1)
            def _(buf=buf, lsem=lsem, cb=cb):
                pltpu.async_copy(
                    u_hbm.at[pl.ds((base_c + cb + 2) * CHUNK, CHUNK)],
                    buf, lsem)
        return carry

    lax.fori_loop(0, HPAIRS, pair, 0)
    plsc.subcore_barrier()
    ob = ub0.shape[0]
    for k in range(NPT // ob):
        off = s * NPT + k * ob
        pltpu.sync_copy(agg_sh.at[pl.ds(off, ob)], ub0)
        pltpu.sync_copy(ub0, out_hbm.at[c].at[pl.ds(off, ob)])


def _proj_body(node_ref, ws_ref, wr_ref, ps_ref, pr_ref):
    x = node_ref[...]
    ps_ref[...] = jnp.dot(x, ws_ref[...], preferred_element_type=jnp.float32)
    pr_ref[...] = jnp.dot(x, wr_ref[...], preferred_element_type=jnp.float32)


def _edge_body(gs_ref, gr_ref, ef_ref, we_ref, we2_ref, b1_ref, b2_ref,
               u_ref, ne_ref):
    ef = ef_ref[...]
    x = gs_ref[...] + gr_ref[...] + b1_ref[...]
    x = x + jnp.dot(ef, we_ref[...], preferred_element_type=jnp.float32)
    h = jnp.maximum(x, 0.0)
    u = jnp.dot(h, we2_ref[...], preferred_element_type=jnp.float32) + b2_ref[...]
    u_ref[...] = u
    ne_ref[...] = u + ef


def _node_body(nf_ref, p0_ref, p1_ref, w1a_ref, w1b_ref, w2_ref, b1_ref,
               b2_ref, out_ref):
    nf = nf_ref[...]
    agg = p0_ref[...] + p1_ref[...]
    x = (jnp.dot(nf, w1a_ref[...], preferred_element_type=jnp.float32)
         + jnp.dot(agg, w1b_ref[...], preferred_element_type=jnp.float32)
         + b1_ref[...])
    h = jnp.maximum(x, 0.0)
    out_ref[...] = (jnp.dot(h, w2_ref[...], preferred_element_type=jnp.float32)
                    + b2_ref[...] + nf)


_BN = 1024
_BNM = 1000
_BE = 2000


def _full(i):
    return (0, 0)


def _rowblk(i):
    return (i, 0)


def kernel(node_features, edge_features, senders, receivers,
           We1, be1, We2, be2, Wn1, bn1, Wn2, bn2):
    f32 = jnp.float32
    pad_e = E_PAD - E
    ar = jnp.arange(pad_e, dtype=jnp.int32)
    sidx = jnp.concatenate([senders.astype(jnp.int32),
                            ar % N_PAD]).reshape(ROWS2, CHUNK)
    ridx = jnp.concatenate([receivers.astype(jnp.int32),
                            N + ar % (N_PAD - N)]).reshape(ROWS2, CHUNK)
    Ws, Wr, We = We1[:D], We1[D:2 * D], We1[2 * D:]
    Wn1a, Wn1b = Wn1[:D], Wn1[D:]
    b_e1, b_e2 = be1.reshape(1, H), be2.reshape(1, D)
    b_n1, b_n2 = bn1.reshape(1, H), bn2.reshape(1, D)

    ps, pr = pl.pallas_call(
        _proj_body,
        grid=(N_PAD // _BN,),
        in_specs=[pl.BlockSpec((_BN, D), _rowblk),
                  pl.BlockSpec((D, D), _full),
                  pl.BlockSpec((D, D), _full)],
        out_specs=[pl.BlockSpec((_BN, D), _rowblk)] * 2,
        out_shape=[jax.ShapeDtypeStruct((N_PAD, D), f32)] * 2,
    )(node_features, Ws, Wr)

    gather = pl.kernel(
        _gather_body,
        out_type=[jax.ShapeDtypeStruct((E_PAD, D), f32)] * 2,
        mesh=_mesh(),
        scratch_types=[pltpu.VMEM((CPW, CHUNK), jnp.int32),
                       pltpu.VMEM((2 * CHUNK, D), f32),
                       pltpu.VMEM((2 * CHUNK, D), f32),
                       pltpu.SemaphoreType.DMA,
                       pltpu.SemaphoreType.DMA,
                       pltpu.SemaphoreType.DMA],
    )
    gs, gr = gather(ps, pr, sidx, ridx)

    u, new_edge = pl.pallas_call(
        _edge_body,
        grid=(E // _BE,),
        in_specs=[pl.BlockSpec((_BE, D), _rowblk),
                  pl.BlockSpec((_BE, D), _rowblk),
                  pl.BlockSpec((_BE, D), _rowblk),
                  pl.BlockSpec((D, H), _full),
                  pl.BlockSpec((H, D), _full),
                  pl.BlockSpec((1, H), _full),
                  pl.BlockSpec((1, D), _full)],
        out_specs=[pl.BlockSpec((_BE, D), _rowblk)] * 2,
        out_shape=[jax.ShapeDtypeStruct((E_PAD, D), f32),
                   jax.ShapeDtypeStruct((E, D), f32)],
    )(gs, gr, edge_features, We, We2, b_e1, b_e2)

    scatter = pl.kernel(
        _scatter_body,
        out_type=jax.ShapeDtypeStruct((NC, N_PAD, D), f32),
        mesh=_mesh(),
        scratch_types=[pltpu.VMEM((CPW, CHUNK), jnp.int32),
                       pltpu.VMEM((CHUNK, D), f32),
                       pltpu.VMEM((CHUNK, D), f32),
                       pltpu.VMEM_SHARED((N_PAD, D), f32),
                       pltpu.SemaphoreType.DMA,
                       pltpu.SemaphoreType.DMA],
    )
    parts = scatter(u, ridx)
    p0, p1 = parts[0, :N], parts[1, :N]

    new_node = pl.pallas_call(
        _node_body,
        grid=(N // _BNM,),
        in_specs=[pl.BlockSpec((_BNM, D), _rowblk),
                  pl.BlockSpec((_BNM, D), _rowblk),
                  pl.BlockSpec((_BNM, D), _rowblk),
                  pl.BlockSpec((D, H), _full),
                  pl.BlockSpec((D, H), _full),
                  pl.BlockSpec((H, D), _full),
                  pl.BlockSpec((1, H), _full),
                  pl.BlockSpec((1, D), _full)],
        out_specs=pl.BlockSpec((_BNM, D), _rowblk),
        out_shape=jax.ShapeDtypeStruct((N, D), f32),
    )(node_features, p0, p1, Wn1a, Wn1b, Wn2, b_n1, b_n2)

    return new_node, new_edge
